# baseline (speedup 1.0000x reference)
import functools

import jax
import jax.numpy as jnp
from jax import lax
from jax.experimental import pallas as pl
from jax.experimental.pallas import tpu as pltpu
from jax.experimental.pallas import tpu_sc as plsc

N_CORES = 2
N_SUBCORES = 16
N_TILES = N_CORES * N_SUBCORES
LANES = 16
CHUNK = 128
BM = 512
CORE0_FRAC = 0.40


def _proj_body(x_ref, wl_ref, wr_ref, y_ref, xr_ref):
    x = x_ref[...]
    y_ref[...] = jnp.dot(x, wl_ref[...], preferred_element_type=jnp.float32)
    xr_ref[...] = jnp.dot(x, wr_ref[...], preferred_element_type=jnp.float32)


def _mid_body(acc_ref, cnt_ref, xr_ref, bl_ref, wl2_ref, wr2_ref, y2_ref, hr_ref):
    agg = acc_ref[0] + acc_ref[1]
    cnt = cnt_ref[0, :, 0:1] + cnt_ref[1, :, 0:1]
    agg = agg / jnp.maximum(cnt, 1.0)
    pre = agg + bl_ref[...] + xr_ref[...]
    norm = jnp.sqrt(jnp.sum(pre * pre, axis=1, keepdims=True))
    h = pre / jnp.maximum(norm, 1e-12)
    h = jnp.maximum(h, 0.0)
    y2_ref[...] = jnp.dot(h, wl2_ref[...], preferred_element_type=jnp.float32)
    hr_ref[...] = jnp.dot(h, wr2_ref[...], preferred_element_type=jnp.float32)


def _out_body(acc_ref, cnt_ref, hr_ref, bl_ref, out_ref):
    agg = acc_ref[0] + acc_ref[1]
    cnt = cnt_ref[0, :, 0:1] + cnt_ref[1, :, 0:1]
    agg = agg / jnp.maximum(cnt, 1.0)
    pre = agg + bl_ref[...] + hr_ref[...]
    norm = jnp.sqrt(jnp.sum(pre * pre, axis=1, keepdims=True))
    o = pre / jnp.maximum(norm, 1e-12)
    z = o - jnp.max(o, axis=1, keepdims=True)
    out_ref[...] = z - jnp.log(jnp.sum(jnp.exp(z), axis=1, keepdims=True))


def _make_sc_segment_sum(n_pad, width, q0, q1, with_count):
    rows_per_tile = n_pad // N_SUBCORES
    mesh = plsc.VectorSubcoreMesh(core_axis_name="c", subcore_axis_name="s")
    q_max = max(q0, q1)
    nb = min(8, min(q0, q1))

    out_type = [jax.ShapeDtypeStruct((N_CORES, n_pad, width), jnp.float32)]
    scratch = [
        pltpu.VMEM((q_max, CHUNK), jnp.int32),
        pltpu.VMEM((q_max, CHUNK), jnp.int32),
        pltpu.VMEM((nb, CHUNK, width), jnp.float32),
        pltpu.VMEM_SHARED((n_pad, width), jnp.float32),
        pltpu.SemaphoreType.DMA((nb,)),
        pltpu.SemaphoreType.DMA((nb,)),
    ]
    if with_count:
        out_type.append(jax.ShapeDtypeStruct((N_CORES, n_pad, LANES), jnp.float32))
        scratch += [
            pltpu.VMEM((CHUNK, LANES), jnp.float32),
            pltpu.VMEM_SHARED((n_pad, LANES), jnp.float32),
            pltpu.SemaphoreType.DMA((nb,)),
        ]

    def body(*refs):
        if with_count:
            (src_hbm, dst_hbm, tbl_hbm, z_hbm, zc_hbm, ones_hbm,
             acc_out, cnt_out,
             src_v, dst_v, msg_v, acc_sh, gsem, ssem, ones_v, cnt_sh, csem) = refs
        else:
            (src_hbm, dst_hbm, tbl_hbm, z_hbm,
             acc_out,
             src_v, dst_v, msg_v, acc_sh, gsem, ssem) = refs
        c = lax.axis_index("c")
        s = lax.axis_index("s")
        q_loc = lax.select(c == 0, q0, q1)
        start = lax.select(c == 0, s * q0, N_SUBCORES * q0 + s * q1)
        r0 = s * rows_per_tile
        rows = pl.ds(r0, rows_per_tile)
        pltpu.sync_copy(z_hbm.at[rows], acc_sh.at[rows])
        if with_count:
            pltpu.sync_copy(zc_hbm.at[rows], cnt_sh.at[rows])
            pltpu.sync_copy(ones_hbm, ones_v)
        pltpu.sync_copy(src_hbm.at[pl.ds(start, q_max)], src_v)
        pltpu.sync_copy(dst_hbm.at[pl.ds(start, q_max)], dst_v)
        plsc.subcore_barrier()

        def gather(j, b):
            pltpu.async_copy(tbl_hbm.at[src_v.at[j]], msg_v.at[b], gsem.at[b])

        for b in range(nb):
            gather(b, b)

        def step(j, carry):
            b = lax.rem(j, nb)
            if with_count:
                @pl.when(j >= nb)
                def _():
                    pltpu.make_async_copy(
                        ones_v, cnt_sh.at[dst_v.at[j]], csem.at[b]).wait()
            pltpu.make_async_copy(
                tbl_hbm.at[src_v.at[j]], msg_v.at[b], gsem.at[b]).wait()
            pltpu.async_copy(msg_v.at[b], acc_sh.at[dst_v.at[j]], ssem.at[b],
                             add=True)
            if with_count:
                pltpu.async_copy(ones_v, cnt_sh.at[dst_v.at[j]], csem.at[b],
                                 add=True)

            @pl.when(j + nb < q_loc)
            def _():
                pltpu.make_async_copy(
                    msg_v.at[b], acc_sh.at[dst_v.at[j]], ssem.at[b]).wait()
                gather(j + nb, b)

            return carry

        lax.fori_loop(0, q_loc, step, 0)
        for b in range(nb):
            pltpu.make_async_copy(
                msg_v.at[b], acc_sh.at[dst_v.at[0]], ssem.at[b]).wait()
            if with_count:
                pltpu.make_async_copy(
                    ones_v, cnt_sh.at[dst_v.at[0]], csem.at[b]).wait()
        plsc.subcore_barrier()
        pltpu.sync_copy(acc_sh.at[rows], acc_out.at[c, rows])
        if with_count:
            pltpu.sync_copy(cnt_sh.at[rows], cnt_out.at[c, rows])

    return pl.kernel(
        body, out_type=out_type, mesh=mesh, scratch_types=scratch,
        compiler_params=pltpu.CompilerParams(use_tc_tiling_on_sc=False))


def _tc_proj(x_pad, Wl, Wr, n_pad, in_ch, out_w):
    grid = (n_pad // BM,)
    return pl.pallas_call(
        _proj_body,
        grid=grid,
        in_specs=[
            pl.BlockSpec((BM, in_ch), lambda i: (i, 0)),
            pl.BlockSpec((in_ch, out_w), lambda i: (0, 0)),
            pl.BlockSpec((in_ch, out_w), lambda i: (0, 0)),
        ],
        out_specs=[
            pl.BlockSpec((BM, out_w), lambda i: (i, 0)),
            pl.BlockSpec((BM, out_w), lambda i: (i, 0)),
        ],
        out_shape=[
            jax.ShapeDtypeStruct((n_pad, out_w), jnp.float32),
            jax.ShapeDtypeStruct((n_pad, out_w), jnp.float32),
        ],
    )(x_pad, Wl, Wr)


def _tc_mid(acc, cnt, xr, bl, Wl2, Wr2, n_pad, hid, out_ch):
    grid = (n_pad // BM,)
    return pl.pallas_call(
        _mid_body,
        grid=grid,
        in_specs=[
            pl.BlockSpec((N_CORES, BM, hid), lambda i: (0, i, 0)),
            pl.BlockSpec((N_CORES, BM, LANES), lambda i: (0, i, 0)),
            pl.BlockSpec((BM, hid), lambda i: (i, 0)),
            pl.BlockSpec((1, hid), lambda i: (0, 0)),
            pl.BlockSpec((hid, out_ch), lambda i: (0, 0)),
            pl.BlockSpec((hid, out_ch), lambda i: (0, 0)),
        ],
        out_specs=[
            pl.BlockSpec((BM, out_ch), lambda i: (i, 0)),
            pl.BlockSpec((BM, out_ch), lambda i: (i, 0)),
        ],
        out_shape=[
            jax.ShapeDtypeStruct((n_pad, out_ch), jnp.float32),
            jax.ShapeDtypeStruct((n_pad, out_ch), jnp.float32),
        ],
    )(acc, cnt, xr, bl, Wl2, Wr2)


def _tc_out(acc, cnt, hr, bl, n_pad, out_ch):
    grid = (n_pad // BM,)
    return pl.pallas_call(
        _out_body,
        grid=grid,
        in_specs=[
            pl.BlockSpec((N_CORES, BM, out_ch), lambda i: (0, i, 0)),
            pl.BlockSpec((N_CORES, BM, LANES), lambda i: (0, i, 0)),
            pl.BlockSpec((BM, out_ch), lambda i: (i, 0)),
            pl.BlockSpec((1, out_ch), lambda i: (0, 0)),
        ],
        out_specs=pl.BlockSpec((BM, out_ch), lambda i: (i, 0)),
        out_shape=jax.ShapeDtypeStruct((n_pad, out_ch), jnp.float32),
    )(acc, cnt, hr, bl)


def kernel(x, edge_index, Wl1, bl1, Wr1, Wl2, bl2, Wr2):
    n, in_ch = x.shape
    hid = Wl1.shape[1]
    out_ch = Wl2.shape[1]
    e = edge_index.shape[1]

    n_pad = -(-n // (N_SUBCORES * BM // 8)) * (N_SUBCORES * BM // 8)
    n_pad = max(n_pad, N_SUBCORES * 8)
    n_chunks = -(-e // (CHUNK * N_SUBCORES)) * N_SUBCORES
    per_core = n_chunks // N_SUBCORES
    q0 = max(1, int(per_core * CORE0_FRAC))
    q1 = per_core - q0
    q_max = max(q0, q1)
    n_chunk_rows = n_chunks + q_max
    e_pad = n_chunk_rows * CHUNK

    src = edge_index[0].astype(jnp.int32)
    dst = edge_index[1].astype(jnp.int32)
    pad_node = jnp.int32(n_pad - 1)
    src = jnp.concatenate([src, jnp.full((e_pad - e,), pad_node)])
    dst = jnp.concatenate([dst, jnp.full((e_pad - e,), pad_node)])
    src = src.reshape(n_chunk_rows, CHUNK)
    dst = dst.reshape(n_chunk_rows, CHUNK)

    x_pad = jnp.pad(x, ((0, n_pad - n), (0, 0)))

    zeros_hid = jnp.zeros((n_pad, hid), jnp.float32)
    zeros_cnt = jnp.zeros((n_pad, LANES), jnp.float32)
    zeros_out = jnp.zeros((n_pad, out_ch), jnp.float32)
    ones_rows = jnp.ones((CHUNK, LANES), jnp.float32)

    y1, xr1 = _tc_proj(x_pad, Wl1, Wr1, n_pad, in_ch, hid)
    sc1 = _make_sc_segment_sum(n_pad, hid, q0, q1, with_count=True)
    acc1, cnt = sc1(src, dst, y1, zeros_hid, zeros_cnt, ones_rows)
    y2, hr2 = _tc_mid(acc1, cnt, xr1, bl1.reshape(1, hid), Wl2, Wr2,
                      n_pad, hid, out_ch)

    sc2 = _make_sc_segment_sum(n_pad, out_ch, q0, q1, with_count=False)
    (acc2,) = sc2(src, dst, y2, zeros_out)
    out = _tc_out(acc2, cnt, hr2, bl2.reshape(1, out_ch), n_pad, out_ch)
    return out[:n]


# speedup vs baseline: 1.0116x; 1.0116x over previous
import functools

import jax
import jax.numpy as jnp
from jax import lax
from jax.experimental import pallas as pl
from jax.experimental.pallas import tpu as pltpu
from jax.experimental.pallas import tpu_sc as plsc

N_CORES = 2
N_SUBCORES = 16
N_TILES = N_CORES * N_SUBCORES
LANES = 16
CHUNK = 128
BM = 512
CORE0_FRAC = 0.45


def _proj_body(x_ref, wl_ref, wr_ref, y_ref, xr_ref):
    x = x_ref[...]
    y_ref[...] = jnp.dot(x, wl_ref[...], preferred_element_type=jnp.float32)
    xr_ref[...] = jnp.dot(x, wr_ref[...], preferred_element_type=jnp.float32)


def _mid_body(acc_ref, cnt_ref, xr_ref, bl_ref, wl2_ref, wr2_ref, y2_ref, hr_ref):
    agg = acc_ref[0] + acc_ref[1]
    cnt = cnt_ref[0, :, 0:1] + cnt_ref[1, :, 0:1]
    agg = agg / jnp.maximum(cnt, 1.0)
    pre = agg + bl_ref[...] + xr_ref[...]
    norm = jnp.sqrt(jnp.sum(pre * pre, axis=1, keepdims=True))
    h = pre / jnp.maximum(norm, 1e-12)
    h = jnp.maximum(h, 0.0)
    y2_ref[...] = jnp.dot(h, wl2_ref[...], preferred_element_type=jnp.float32)
    hr_ref[...] = jnp.dot(h, wr2_ref[...], preferred_element_type=jnp.float32)


def _out_body(acc_ref, cnt_ref, hr_ref, bl_ref, out_ref):
    agg = acc_ref[0] + acc_ref[1]
    cnt = cnt_ref[0, :, 0:1] + cnt_ref[1, :, 0:1]
    agg = agg / jnp.maximum(cnt, 1.0)
    pre = agg + bl_ref[...] + hr_ref[...]
    norm = jnp.sqrt(jnp.sum(pre * pre, axis=1, keepdims=True))
    o = pre / jnp.maximum(norm, 1e-12)
    z = o - jnp.max(o, axis=1, keepdims=True)
    out_ref[...] = z - jnp.log(jnp.sum(jnp.exp(z), axis=1, keepdims=True))


def _make_sc_segment_sum(n_pad, width, q0, q1, with_count):
    rows_per_tile = n_pad // N_SUBCORES
    mesh = plsc.VectorSubcoreMesh(core_axis_name="c", subcore_axis_name="s")
    q_max = max(q0, q1)
    nb = min(8, min(q0, q1))

    out_type = [jax.ShapeDtypeStruct((N_CORES, n_pad, width), jnp.float32)]
    scratch = [
        pltpu.VMEM((q_max, CHUNK), jnp.int32),
        pltpu.VMEM((q_max, CHUNK), jnp.int32),
        pltpu.VMEM((nb, CHUNK, width), jnp.float32),
        pltpu.VMEM_SHARED((n_pad, width), jnp.float32),
        pltpu.SemaphoreType.DMA((nb,)),
        pltpu.SemaphoreType.DMA((nb,)),
    ]
    if with_count:
        out_type.append(jax.ShapeDtypeStruct((N_CORES, n_pad, LANES), jnp.float32))
        scratch += [
            pltpu.VMEM((CHUNK, LANES), jnp.float32),
            pltpu.VMEM_SHARED((n_pad, LANES), jnp.float32),
            pltpu.SemaphoreType.DMA((nb,)),
        ]

    def body(*refs):
        if with_count:
            (src_hbm, dst_hbm, tbl_hbm, z_hbm, zc_hbm, ones_hbm,
             acc_out, cnt_out,
             src_v, dst_v, msg_v, acc_sh, gsem, ssem, ones_v, cnt_sh, csem) = refs
        else:
            (src_hbm, dst_hbm, tbl_hbm, z_hbm,
             acc_out,
             src_v, dst_v, msg_v, acc_sh, gsem, ssem) = refs
        c = lax.axis_index("c")
        s = lax.axis_index("s")
        q_loc = lax.select(c == 0, q0, q1)
        start = lax.select(c == 0, s * q0, N_SUBCORES * q0 + s * q1)
        r0 = s * rows_per_tile
        rows = pl.ds(r0, rows_per_tile)
        pltpu.sync_copy(z_hbm.at[rows], acc_sh.at[rows])
        if with_count:
            pltpu.sync_copy(zc_hbm.at[rows], cnt_sh.at[rows])
            pltpu.sync_copy(ones_hbm, ones_v)
        pltpu.sync_copy(src_hbm.at[pl.ds(start, q_max)], src_v)
        pltpu.sync_copy(dst_hbm.at[pl.ds(start, q_max)], dst_v)
        plsc.subcore_barrier()

        def gather(j, b):
            pltpu.async_copy(tbl_hbm.at[src_v.at[j]], msg_v.at[b], gsem.at[b])

        for b in range(nb):
            gather(b, b)

        def step(j, carry):
            b = lax.rem(j, nb)
            if with_count:
                @pl.when(j >= nb)
                def _():
                    pltpu.make_async_copy(
                        ones_v, cnt_sh.at[dst_v.at[j]], csem.at[b]).wait()
            pltpu.make_async_copy(
                tbl_hbm.at[src_v.at[j]], msg_v.at[b], gsem.at[b]).wait()
            pltpu.async_copy(msg_v.at[b], acc_sh.at[dst_v.at[j]], ssem.at[b],
                             add=True)
            if with_count:
                pltpu.async_copy(ones_v, cnt_sh.at[dst_v.at[j]], csem.at[b],
                                 add=True)

            @pl.when(j + nb < q_loc)
            def _():
                pltpu.make_async_copy(
                    msg_v.at[b], acc_sh.at[dst_v.at[j]], ssem.at[b]).wait()
                gather(j + nb, b)

            return carry

        lax.fori_loop(0, q_loc, step, 0)
        for b in range(nb):
            pltpu.make_async_copy(
                msg_v.at[b], acc_sh.at[dst_v.at[0]], ssem.at[b]).wait()
            if with_count:
                pltpu.make_async_copy(
                    ones_v, cnt_sh.at[dst_v.at[0]], csem.at[b]).wait()
        plsc.subcore_barrier()
        pltpu.sync_copy(acc_sh.at[rows], acc_out.at[c, rows])
        if with_count:
            pltpu.sync_copy(cnt_sh.at[rows], cnt_out.at[c, rows])

    return pl.kernel(
        body, out_type=out_type, mesh=mesh, scratch_types=scratch,
        compiler_params=pltpu.CompilerParams(use_tc_tiling_on_sc=False))


def _tc_proj(x_pad, Wl, Wr, n_pad, in_ch, out_w):
    grid = (n_pad // BM,)
    return pl.pallas_call(
        _proj_body,
        grid=grid,
        in_specs=[
            pl.BlockSpec((BM, in_ch), lambda i: (i, 0)),
            pl.BlockSpec((in_ch, out_w), lambda i: (0, 0)),
            pl.BlockSpec((in_ch, out_w), lambda i: (0, 0)),
        ],
        out_specs=[
            pl.BlockSpec((BM, out_w), lambda i: (i, 0)),
            pl.BlockSpec((BM, out_w), lambda i: (i, 0)),
        ],
        out_shape=[
            jax.ShapeDtypeStruct((n_pad, out_w), jnp.float32),
            jax.ShapeDtypeStruct((n_pad, out_w), jnp.float32),
        ],
    )(x_pad, Wl, Wr)


def _tc_mid(acc, cnt, xr, bl, Wl2, Wr2, n_pad, hid, out_ch):
    grid = (n_pad // BM,)
    return pl.pallas_call(
        _mid_body,
        grid=grid,
        in_specs=[
            pl.BlockSpec((N_CORES, BM, hid), lambda i: (0, i, 0)),
            pl.BlockSpec((N_CORES, BM, LANES), lambda i: (0, i, 0)),
            pl.BlockSpec((BM, hid), lambda i: (i, 0)),
            pl.BlockSpec((1, hid), lambda i: (0, 0)),
            pl.BlockSpec((hid, out_ch), lambda i: (0, 0)),
            pl.BlockSpec((hid, out_ch), lambda i: (0, 0)),
        ],
        out_specs=[
            pl.BlockSpec((BM, out_ch), lambda i: (i, 0)),
            pl.BlockSpec((BM, out_ch), lambda i: (i, 0)),
        ],
        out_shape=[
            jax.ShapeDtypeStruct((n_pad, out_ch), jnp.float32),
            jax.ShapeDtypeStruct((n_pad, out_ch), jnp.float32),
        ],
    )(acc, cnt, xr, bl, Wl2, Wr2)


def _tc_out(acc, cnt, hr, bl, n_pad, out_ch):
    grid = (n_pad // BM,)
    return pl.pallas_call(
        _out_body,
        grid=grid,
        in_specs=[
            pl.BlockSpec((N_CORES, BM, out_ch), lambda i: (0, i, 0)),
            pl.BlockSpec((N_CORES, BM, LANES), lambda i: (0, i, 0)),
            pl.BlockSpec((BM, out_ch), lambda i: (i, 0)),
            pl.BlockSpec((1, out_ch), lambda i: (0, 0)),
        ],
        out_specs=pl.BlockSpec((BM, out_ch), lambda i: (i, 0)),
        out_shape=jax.ShapeDtypeStruct((n_pad, out_ch), jnp.float32),
    )(acc, cnt, hr, bl)


def kernel(x, edge_index, Wl1, bl1, Wr1, Wl2, bl2, Wr2):
    n, in_ch = x.shape
    hid = Wl1.shape[1]
    out_ch = Wl2.shape[1]
    e = edge_index.shape[1]

    n_pad = -(-n // (N_SUBCORES * BM // 8)) * (N_SUBCORES * BM // 8)
    n_pad = max(n_pad, N_SUBCORES * 8)
    n_chunks = -(-e // (CHUNK * N_SUBCORES)) * N_SUBCORES
    per_core = n_chunks // N_SUBCORES
    q0 = max(1, int(per_core * CORE0_FRAC))
    q1 = per_core - q0
    q_max = max(q0, q1)
    n_chunk_rows = n_chunks + q_max
    e_pad = n_chunk_rows * CHUNK

    src = edge_index[0].astype(jnp.int32)
    dst = edge_index[1].astype(jnp.int32)
    pad_node = jnp.int32(n_pad - 1)
    src = jnp.concatenate([src, jnp.full((e_pad - e,), pad_node)])
    dst = jnp.concatenate([dst, jnp.full((e_pad - e,), pad_node)])
    src = src.reshape(n_chunk_rows, CHUNK)
    dst = dst.reshape(n_chunk_rows, CHUNK)

    x_pad = jnp.pad(x, ((0, n_pad - n), (0, 0)))

    zeros_hid = jnp.zeros((n_pad, hid), jnp.float32)
    zeros_cnt = jnp.zeros((n_pad, LANES), jnp.float32)
    zeros_out = jnp.zeros((n_pad, out_ch), jnp.float32)
    ones_rows = jnp.ones((CHUNK, LANES), jnp.float32)

    y1, xr1 = _tc_proj(x_pad, Wl1, Wr1, n_pad, in_ch, hid)
    sc1 = _make_sc_segment_sum(n_pad, hid, q0, q1, with_count=True)
    acc1, cnt = sc1(src, dst, y1, zeros_hid, zeros_cnt, ones_rows)
    y2, hr2 = _tc_mid(acc1, cnt, xr1, bl1.reshape(1, hid), Wl2, Wr2,
                      n_pad, hid, out_ch)

    sc2 = _make_sc_segment_sum(n_pad, out_ch, q0, q1, with_count=False)
    (acc2,) = sc2(src, dst, y2, zeros_out)
    out = _tc_out(acc2, cnt, hr2, bl2.reshape(1, out_ch), n_pad, out_ch)
    return out[:n]


# speedup vs baseline: 1.1159x; 1.1032x over previous
import functools

import jax
import jax.numpy as jnp
from jax import lax
from jax.experimental import pallas as pl
from jax.experimental.pallas import tpu as pltpu
from jax.experimental.pallas import tpu_sc as plsc

N_CORES = 2
N_SUBCORES = 16
N_TILES = N_CORES * N_SUBCORES
LANES = 16
CHUNK = 128
BM = 512
MIXW = 128
CORE0_FRAC = 0.45


def _proj_body(x_ref, wl_ref, wr_ref, y_ref, xr_ref):
    x = x_ref[...]
    y_ref[...] = jnp.dot(x, wl_ref[...], preferred_element_type=jnp.float32)
    xr_ref[...] = jnp.dot(x, wr_ref[...], preferred_element_type=jnp.float32)


def _mid_body(mix_ref, xr_ref, bl_ref, wl2_ref, wr2_ref, y2_ref, hr_ref):
    hid = xr_ref.shape[1]
    m = mix_ref[...]
    agg = m[:, 0:hid] + m[:, hid:2 * hid]
    cnt = m[:, 2 * hid:2 * hid + 1] + m[:, 2 * hid + LANES:2 * hid + LANES + 1]
    agg = agg / jnp.maximum(cnt, 1.0)
    pre = agg + bl_ref[...] + xr_ref[...]
    norm = jnp.sqrt(jnp.sum(pre * pre, axis=1, keepdims=True))
    h = pre / jnp.maximum(norm, 1e-12)
    h = jnp.maximum(h, 0.0)
    y2_ref[...] = jnp.dot(h, wl2_ref[...], preferred_element_type=jnp.float32)
    hr_ref[...] = jnp.dot(h, wr2_ref[...], preferred_element_type=jnp.float32)


def _out_body(mix2_ref, mix1_ref, hr_ref, bl_ref, out_ref):
    out_ch = hr_ref.shape[1]
    hid = mix1_ref.shape[1] // 4
    m2 = mix2_ref[...]
    m1 = mix1_ref[...]
    agg = m2[:, 0:out_ch] + m2[:, out_ch:2 * out_ch]
    cnt = m1[:, 2 * hid:2 * hid + 1] + m1[:, 2 * hid + LANES:2 * hid + LANES + 1]
    agg = agg / jnp.maximum(cnt, 1.0)
    pre = agg + bl_ref[...] + hr_ref[...]
    norm = jnp.sqrt(jnp.sum(pre * pre, axis=1, keepdims=True))
    o = pre / jnp.maximum(norm, 1e-12)
    z = o - jnp.max(o, axis=1, keepdims=True)
    out_ref[...] = z - jnp.log(jnp.sum(jnp.exp(z), axis=1, keepdims=True))


def _make_sc_segment_sum(n_pad, width, q0, q1, with_count):
    rows_per_tile = n_pad // N_SUBCORES
    mesh = plsc.VectorSubcoreMesh(core_axis_name="c", subcore_axis_name="s")
    q_max = max(q0, q1)
    nb = min(8, min(q0, q1))

    out_type = jax.ShapeDtypeStruct((n_pad, MIXW), jnp.float32)
    scratch = [
        pltpu.VMEM((q_max, CHUNK), jnp.int32),
        pltpu.VMEM((q_max, CHUNK), jnp.int32),
        pltpu.VMEM((nb, CHUNK, width), jnp.float32),
        pltpu.VMEM_SHARED((n_pad, width), jnp.float32),
        pltpu.SemaphoreType.DMA((nb,)),
        pltpu.SemaphoreType.DMA((nb,)),
    ]
    if with_count:
        scratch += [
            pltpu.VMEM((CHUNK, LANES), jnp.float32),
            pltpu.VMEM_SHARED((n_pad, LANES), jnp.float32),
            pltpu.SemaphoreType.DMA((nb,)),
        ]

    def body(*refs):
        if with_count:
            (src_hbm, dst_hbm, tbl_hbm, mix_out,
             src_v, dst_v, msg_v, acc_sh, gsem, ssem, ones_v, cnt_sh, csem) = refs
        else:
            (src_hbm, dst_hbm, tbl_hbm, mix_out,
             src_v, dst_v, msg_v, acc_sh, gsem, ssem) = refs
        c = lax.axis_index("c")
        s = lax.axis_index("s")
        q_loc = lax.select(c == 0, q0, q1)
        start = lax.select(c == 0, s * q0, N_SUBCORES * q0 + s * q1)
        r0 = s * rows_per_tile
        rows = pl.ds(r0, rows_per_tile)

        zv = jnp.zeros((LANES,), jnp.float32)

        def zrow(i, carry):
            for k in range(width // LANES):
                msg_v[0, i, pl.ds(k * LANES, LANES)] = zv
            return carry

        lax.fori_loop(0, CHUNK, zrow, 0)
        for k in range(rows_per_tile // CHUNK):
            pltpu.sync_copy(msg_v.at[0],
                            acc_sh.at[pl.ds(r0 + k * CHUNK, CHUNK)])
        if with_count:
            def onerow(i, carry):
                ones_v[i, pl.ds(0, LANES)] = zv
                return carry

            lax.fori_loop(0, CHUNK, onerow, 0)
            for k in range(rows_per_tile // CHUNK):
                pltpu.sync_copy(ones_v,
                                cnt_sh.at[pl.ds(r0 + k * CHUNK, CHUNK)])
            ov = jnp.ones((LANES,), jnp.float32)

            def onerow2(i, carry):
                ones_v[i, pl.ds(0, LANES)] = ov
                return carry

            lax.fori_loop(0, CHUNK, onerow2, 0)
        pltpu.sync_copy(src_hbm.at[pl.ds(start, q_max)], src_v)
        pltpu.sync_copy(dst_hbm.at[pl.ds(start, q_max)], dst_v)
        plsc.subcore_barrier()

        def gather(j, b):
            pltpu.async_copy(tbl_hbm.at[src_v.at[j]], msg_v.at[b], gsem.at[b])

        for b in range(nb):
            gather(b, b)

        def step(j, carry):
            b = lax.rem(j, nb)
            if with_count:
                @pl.when(j >= nb)
                def _():
                    pltpu.make_async_copy(
                        ones_v, cnt_sh.at[dst_v.at[j]], csem.at[b]).wait()
            pltpu.make_async_copy(
                tbl_hbm.at[src_v.at[j]], msg_v.at[b], gsem.at[b]).wait()
            pltpu.async_copy(msg_v.at[b], acc_sh.at[dst_v.at[j]], ssem.at[b],
                             add=True)
            if with_count:
                pltpu.async_copy(ones_v, cnt_sh.at[dst_v.at[j]], csem.at[b],
                                 add=True)

            @pl.when(j + nb < q_loc)
            def _():
                pltpu.make_async_copy(
                    msg_v.at[b], acc_sh.at[dst_v.at[j]], ssem.at[b]).wait()
                gather(j + nb, b)

            return carry

        lax.fori_loop(0, q_loc, step, 0)
        for b in range(nb):
            pltpu.make_async_copy(
                msg_v.at[b], acc_sh.at[dst_v.at[0]], ssem.at[b]).wait()
            if with_count:
                pltpu.make_async_copy(
                    ones_v, cnt_sh.at[dst_v.at[0]], csem.at[b]).wait()
        plsc.subcore_barrier()
        pltpu.sync_copy(acc_sh.at[rows],
                        mix_out.at[rows, pl.ds(c * width, width)])
        if with_count:
            pltpu.sync_copy(cnt_sh.at[rows],
                            mix_out.at[rows, pl.ds(2 * width + c * LANES, LANES)])

    return pl.kernel(
        body, out_type=out_type, mesh=mesh, scratch_types=scratch,
        compiler_params=pltpu.CompilerParams(use_tc_tiling_on_sc=False))


def _tc_proj(x_pad, Wl, Wr, n_pad, in_ch, out_w):
    grid = (n_pad // BM,)
    return pl.pallas_call(
        _proj_body,
        grid=grid,
        in_specs=[
            pl.BlockSpec((BM, in_ch), lambda i: (i, 0)),
            pl.BlockSpec((in_ch, out_w), lambda i: (0, 0)),
            pl.BlockSpec((in_ch, out_w), lambda i: (0, 0)),
        ],
        out_specs=[
            pl.BlockSpec((BM, out_w), lambda i: (i, 0)),
            pl.BlockSpec((BM, out_w), lambda i: (i, 0)),
        ],
        out_shape=[
            jax.ShapeDtypeStruct((n_pad, out_w), jnp.float32),
            jax.ShapeDtypeStruct((n_pad, out_w), jnp.float32),
        ],
    )(x_pad, Wl, Wr)


def _tc_mid(mix, xr, bl, Wl2, Wr2, n_pad, hid, out_ch):
    grid = (n_pad // BM,)
    return pl.pallas_call(
        _mid_body,
        grid=grid,
        in_specs=[
            pl.BlockSpec((BM, MIXW), lambda i: (i, 0)),
            pl.BlockSpec((BM, hid), lambda i: (i, 0)),
            pl.BlockSpec((1, hid), lambda i: (0, 0)),
            pl.BlockSpec((hid, out_ch), lambda i: (0, 0)),
            pl.BlockSpec((hid, out_ch), lambda i: (0, 0)),
        ],
        out_specs=[
            pl.BlockSpec((BM, out_ch), lambda i: (i, 0)),
            pl.BlockSpec((BM, out_ch), lambda i: (i, 0)),
        ],
        out_shape=[
            jax.ShapeDtypeStruct((n_pad, out_ch), jnp.float32),
            jax.ShapeDtypeStruct((n_pad, out_ch), jnp.float32),
        ],
    )(mix, xr, bl, Wl2, Wr2)


def _tc_out(mix2, mix1, hr, bl, n_pad, out_ch):
    grid = (n_pad // BM,)
    return pl.pallas_call(
        _out_body,
        grid=grid,
        in_specs=[
            pl.BlockSpec((BM, MIXW), lambda i: (i, 0)),
            pl.BlockSpec((BM, MIXW), lambda i: (i, 0)),
            pl.BlockSpec((BM, out_ch), lambda i: (i, 0)),
            pl.BlockSpec((1, out_ch), lambda i: (0, 0)),
        ],
        out_specs=pl.BlockSpec((BM, out_ch), lambda i: (i, 0)),
        out_shape=jax.ShapeDtypeStruct((n_pad, out_ch), jnp.float32),
    )(mix2, mix1, hr, bl)


def kernel(x, edge_index, Wl1, bl1, Wr1, Wl2, bl2, Wr2):
    n, in_ch = x.shape
    hid = Wl1.shape[1]
    out_ch = Wl2.shape[1]
    e = edge_index.shape[1]

    n_pad = -(-n // (N_SUBCORES * BM // 8)) * (N_SUBCORES * BM // 8)
    n_pad = max(n_pad, N_SUBCORES * 8)
    n_chunks = -(-e // (CHUNK * N_SUBCORES)) * N_SUBCORES
    per_core = n_chunks // N_SUBCORES
    q0 = max(1, int(per_core * CORE0_FRAC))
    q1 = per_core - q0
    q_max = max(q0, q1)
    n_chunk_rows = -(-(n_chunks + q_max) // 8) * 8
    e_pad = n_chunk_rows * CHUNK

    src = edge_index[0].astype(jnp.int32)
    dst = edge_index[1].astype(jnp.int32)
    pad_node = jnp.int32(n_pad - 1)
    src = jnp.concatenate([src, jnp.full((e_pad - e,), pad_node)])
    dst = jnp.concatenate([dst, jnp.full((e_pad - e,), pad_node)])
    src = src.reshape(n_chunk_rows, CHUNK)
    dst = dst.reshape(n_chunk_rows, CHUNK)

    x_pad = jnp.pad(x, ((0, n_pad - n), (0, 0)))

    y1, xr1 = _tc_proj(x_pad, Wl1, Wr1, n_pad, in_ch, hid)
    sc1 = _make_sc_segment_sum(n_pad, hid, q0, q1, with_count=True)
    mix1 = sc1(src, dst, y1)
    y2, hr2 = _tc_mid(mix1, xr1, bl1.reshape(1, hid), Wl2, Wr2,
                      n_pad, hid, out_ch)

    sc2 = _make_sc_segment_sum(n_pad, out_ch, q0, q1, with_count=False)
    mix2 = sc2(src, dst, y2)
    out = _tc_out(mix2, mix1, hr2, bl2.reshape(1, out_ch), n_pad, out_ch)
    return out[:n]
